# Initial kernel scaffold; baseline (speedup 1.0000x reference)
import jax
import jax.numpy as jnp
from jax.experimental import pallas as pl


def kernel(embedding, edges, W_gcn, b_gcn, W_cls, b_cls):
    raise NotImplementedError("write your pallas kernel here")


# speedup vs baseline: 168.4059x; 168.4059x over previous
import functools

import jax
import jax.numpy as jnp
from jax import lax
from jax.experimental import pallas as pl
from jax.experimental.pallas import tpu as pltpu
from jax.experimental.pallas import tpu_sc as plsc

N_NODES = 10000
N_EDGES = 320000
NC = 2
NS = 16
NW = NC * NS
EPW = N_EDGES // NW
L = 16


def _sc_histograms(src, dst):
    mesh = plsc.VectorSubcoreMesh(
        core_axis_name="c", subcore_axis_name="s", num_cores=NC, num_subcores=NS
    )

    @functools.partial(
        pl.kernel,
        mesh=mesh,
        compiler_params=pltpu.CompilerParams(needs_layout_passes=False),
        out_type=[
            jax.ShapeDtypeStruct((NW, N_NODES), jnp.float32),
            jax.ShapeDtypeStruct((NW, N_NODES), jnp.float32),
        ],
        scratch_types=[
            pltpu.VMEM((EPW,), jnp.int32),
            pltpu.VMEM((EPW,), jnp.int32),
            pltpu.VMEM((N_NODES,), jnp.float32),
            pltpu.VMEM((N_NODES,), jnp.float32),
        ],
    )
    def hist_kernel(src_hbm, dst_hbm, deg_out, c0_out, src_v, dst_v, deg_v, c0_v):
        wid = lax.axis_index("s") * NC + lax.axis_index("c")
        base = wid * EPW
        pltpu.sync_copy(src_hbm.at[pl.ds(base, EPW)], src_v)
        pltpu.sync_copy(dst_hbm.at[pl.ds(base, EPW)], dst_v)

        zeros = jnp.zeros((L,), jnp.float32)

        def zero_body(i, carry):
            deg_v[pl.ds(i * L, L)] = zeros
            c0_v[pl.ds(i * L, L)] = zeros
            return carry

        lax.fori_loop(0, N_NODES // L, zero_body, 0)

        ones = jnp.ones((L,), jnp.float32)

        def edge_body(i, carry):
            s = src_v[pl.ds(i * L, L)]
            d = dst_v[pl.ds(i * L, L)]
            plsc.addupdate_scatter(deg_v, [d], ones)
            plsc.addupdate_scatter(c0_v, [s], ones, mask=(d == 0))
            return carry

        lax.fori_loop(0, EPW // L, edge_body, 0)

        pltpu.sync_copy(deg_v, deg_out.at[wid])
        pltpu.sync_copy(c0_v, c0_out.at[wid])

    return hist_kernel(src, dst)


def _tc_body(deg_ref, c0_ref, x_ref, wg_ref, bg_ref, wc_ref, bc_ref, out_ref):
    deg = jnp.sum(deg_ref[...], axis=0, keepdims=True) + 1.0
    dis = lax.rsqrt(deg)
    c0 = jnp.sum(c0_ref[...], axis=0, keepdims=True)
    col = lax.broadcasted_iota(jnp.int32, (1, N_NODES), 1)
    is0 = col == 0
    dis0 = jnp.sum(jnp.where(is0, dis, 0.0))
    u = c0 * dis + jnp.where(is0, dis0, 0.0)
    s = jnp.dot(u, x_ref[...], preferred_element_type=jnp.float32)
    h = dis0 * jnp.dot(s, wg_ref[...], preferred_element_type=jnp.float32)
    h = h + bg_ref[...]
    out_ref[...] = jnp.dot(h, wc_ref[...], preferred_element_type=jnp.float32) + bc_ref[...]


def _tc_combine(deg_parts, c0_parts, x, W_gcn, b_gcn, W_cls, b_cls):
    return pl.pallas_call(
        _tc_body,
        out_shape=jax.ShapeDtypeStruct((1, W_cls.shape[1]), jnp.float32),
    )(deg_parts, c0_parts, x, W_gcn, b_gcn, W_cls, b_cls)


def kernel(embedding, edges, W_gcn, b_gcn, W_cls, b_cls):
    e = edges.astype(jnp.int32)
    src = e[0]
    dst = e[1]
    deg_parts, c0_parts = _sc_histograms(src, dst)
    return _tc_combine(
        deg_parts,
        c0_parts,
        embedding,
        W_gcn,
        b_gcn.reshape(1, -1),
        W_cls,
        b_cls.reshape(1, -1),
    )


# speedup vs baseline: 215.4375x; 1.2793x over previous
import functools

import jax
import jax.numpy as jnp
from jax import lax
from jax.experimental import pallas as pl
from jax.experimental.pallas import tpu as pltpu
from jax.experimental.pallas import tpu_sc as plsc

N_NODES = 10000
N_EDGES = 320000
NC = 2
NS = 16
NW = NC * NS
EPW = N_EDGES // NW
L = 16


def _sc_histograms(edges):
    mesh = plsc.VectorSubcoreMesh(
        core_axis_name="c", subcore_axis_name="s", num_cores=NC, num_subcores=NS
    )

    @functools.partial(
        pl.kernel,
        mesh=mesh,
        compiler_params=pltpu.CompilerParams(
            needs_layout_passes=False, skip_device_barrier=True
        ),
        out_type=[
            jax.ShapeDtypeStruct((NW, N_NODES), jnp.float32),
            jax.ShapeDtypeStruct((NW, N_NODES), jnp.float32),
        ],
        scratch_types=[
            pltpu.VMEM((EPW,), jnp.int32),
            pltpu.VMEM((EPW,), jnp.int32),
            pltpu.VMEM((N_NODES,), jnp.float32),
            pltpu.VMEM((N_NODES,), jnp.float32),
        ],
    )
    def hist_kernel(edges_hbm, deg_out, c0_out, src_v, dst_v, deg_v, c0_v):
        wid = lax.axis_index("s") * NC + lax.axis_index("c")
        base = wid * EPW
        pltpu.sync_copy(edges_hbm.at[pl.ds(base, EPW)], src_v)
        pltpu.sync_copy(edges_hbm.at[pl.ds(N_EDGES + base, EPW)], dst_v)

        zeros = jnp.zeros((L,), jnp.float32)

        def zero_body(i, carry):
            deg_v[pl.ds(i * L, L)] = zeros
            c0_v[pl.ds(i * L, L)] = zeros
            return carry

        lax.fori_loop(0, N_NODES // L, zero_body, 0)

        ones = jnp.ones((L,), jnp.float32)

        def edge_body(i, carry):
            s = src_v[pl.ds(i * L, L)]
            d = dst_v[pl.ds(i * L, L)]
            plsc.addupdate_scatter(deg_v, [d], ones)
            plsc.addupdate_scatter(c0_v, [s], ones, mask=(d == 0))
            return carry

        lax.fori_loop(0, EPW // L, edge_body, 0)

        pltpu.sync_copy(deg_v, deg_out.at[wid])
        pltpu.sync_copy(c0_v, c0_out.at[wid])

    return hist_kernel(edges)


def _tc_body(deg_ref, c0_ref, x_ref, wg_ref, bg_ref, wc_ref, bc_ref, out_ref):
    deg = jnp.sum(deg_ref[...], axis=0, keepdims=True) + 1.0
    dis = lax.rsqrt(deg)
    c0 = jnp.sum(c0_ref[...], axis=0, keepdims=True)
    col = lax.broadcasted_iota(jnp.int32, (1, N_NODES), 1)
    is0 = col == 0
    dis0 = jnp.sum(jnp.where(is0, dis, 0.0))
    u = c0 * dis + jnp.where(is0, dis0, 0.0)
    s = jnp.dot(u, x_ref[...], preferred_element_type=jnp.float32)
    h = dis0 * jnp.dot(s, wg_ref[...], preferred_element_type=jnp.float32)
    h = h + bg_ref[...]
    out_ref[...] = jnp.dot(h, wc_ref[...], preferred_element_type=jnp.float32) + bc_ref[...]


def _tc_combine(deg_parts, c0_parts, x, W_gcn, b_gcn, W_cls, b_cls):
    return pl.pallas_call(
        _tc_body,
        out_shape=jax.ShapeDtypeStruct((1, W_cls.shape[1]), jnp.float32),
    )(deg_parts, c0_parts, x, W_gcn, b_gcn, W_cls, b_cls)


def kernel(embedding, edges, W_gcn, b_gcn, W_cls, b_cls):
    deg_parts, c0_parts = _sc_histograms(edges.astype(jnp.int32).reshape(-1))
    return _tc_combine(
        deg_parts,
        c0_parts,
        embedding,
        W_gcn,
        b_gcn.reshape(1, -1),
        W_cls,
        b_cls.reshape(1, -1),
    )


# speedup vs baseline: 233.4171x; 1.0835x over previous
import functools

import jax
import jax.numpy as jnp
from jax import lax
from jax.experimental import pallas as pl
from jax.experimental.pallas import tpu as pltpu
from jax.experimental.pallas import tpu_sc as plsc

N_NODES = 10000
N_EDGES = 320000
NC = 2
NS = 16
NW = NC * NS
EPW = N_EDGES // NW
L = 16


def _sc_histograms(edges):
    mesh = plsc.VectorSubcoreMesh(
        core_axis_name="c", subcore_axis_name="s", num_cores=NC, num_subcores=NS
    )

    @functools.partial(
        pl.kernel,
        mesh=mesh,
        compiler_params=pltpu.CompilerParams(
            needs_layout_passes=False, skip_device_barrier=True
        ),
        out_type=[
            jax.ShapeDtypeStruct((NW, N_NODES), jnp.float32),
            jax.ShapeDtypeStruct((NW, N_NODES), jnp.float32),
        ],
        scratch_types=[
            pltpu.VMEM((EPW,), jnp.int32),
            pltpu.VMEM((EPW,), jnp.int32),
            pltpu.VMEM((N_NODES,), jnp.float32),
            pltpu.VMEM((N_NODES,), jnp.float32),
            pltpu.SemaphoreType.DMA,
        ],
    )
    def hist_kernel(edges_hbm, deg_out, c0_out, src_v, dst_v, deg_v, c0_v, sem):
        wid = lax.axis_index("s") * NC + lax.axis_index("c")
        base = wid * EPW
        cp_src = pltpu.async_copy(edges_hbm.at[pl.ds(base, EPW)], src_v, sem)
        cp_dst = pltpu.async_copy(
            edges_hbm.at[pl.ds(N_EDGES + base, EPW)], dst_v, sem
        )

        zeros = jnp.zeros((L,), jnp.float32)

        def zero_body(i, carry):
            for k in range(4):
                deg_v[pl.ds((i * 4 + k) * L, L)] = zeros
                c0_v[pl.ds((i * 4 + k) * L, L)] = zeros
            return carry

        lax.fori_loop(0, N_NODES // (4 * L), zero_body, 0)
        cp_src.wait()
        cp_dst.wait()

        ones = jnp.ones((L,), jnp.float32)

        def edge_body(i, carry):
            s = src_v[pl.ds(i * L, L)]
            d = dst_v[pl.ds(i * L, L)]
            plsc.addupdate_scatter(deg_v, [d], ones)
            plsc.addupdate_scatter(c0_v, [s], ones, mask=(d == 0))
            return carry

        lax.fori_loop(0, EPW // L, edge_body, 0)

        pltpu.sync_copy(deg_v, deg_out.at[wid])
        pltpu.sync_copy(c0_v, c0_out.at[wid])

    return hist_kernel(edges)


def _tc_body(deg_ref, c0_ref, x_ref, wg_ref, bg_ref, wc_ref, bc_ref, out_ref):
    deg = jnp.sum(deg_ref[...], axis=0, keepdims=True) + 1.0
    dis = lax.rsqrt(deg)
    c0 = jnp.sum(c0_ref[...], axis=0, keepdims=True)
    col = lax.broadcasted_iota(jnp.int32, (1, N_NODES), 1)
    is0 = col == 0
    dis0 = jnp.sum(jnp.where(is0, dis, 0.0))
    u = c0 * dis + jnp.where(is0, dis0, 0.0)
    s = jnp.dot(u, x_ref[...], preferred_element_type=jnp.float32)
    h = dis0 * jnp.dot(s, wg_ref[...], preferred_element_type=jnp.float32)
    h = h + bg_ref[...]
    out_ref[...] = jnp.dot(h, wc_ref[...], preferred_element_type=jnp.float32) + bc_ref[...]


def _tc_combine(deg_parts, c0_parts, x, W_gcn, b_gcn, W_cls, b_cls):
    return pl.pallas_call(
        _tc_body,
        out_shape=jax.ShapeDtypeStruct((1, W_cls.shape[1]), jnp.float32),
    )(deg_parts, c0_parts, x, W_gcn, b_gcn, W_cls, b_cls)


def kernel(embedding, edges, W_gcn, b_gcn, W_cls, b_cls):
    deg_parts, c0_parts = _sc_histograms(edges.astype(jnp.int32).reshape(-1))
    return _tc_combine(
        deg_parts,
        c0_parts,
        embedding,
        W_gcn,
        b_gcn.reshape(1, -1),
        W_cls,
        b_cls.reshape(1, -1),
    )
